# baseline (speedup 1.0000x reference)
import functools

import jax
import jax.numpy as jnp
from jax import lax
from jax.experimental import pallas as pl
from jax.experimental.pallas import tpu as pltpu
from jax.experimental.pallas import tpu_sc as plsc

B = 8
N = 20000
K = 17
KEEP = 1000
NV = N // 16
SELV = 63
CAND = N + 32

_i32 = jnp.int32
_u32 = jnp.uint32
_f32 = jnp.float32

_IOTA = lambda: lax.iota(_i32, 16)


def _mono(f):
  b = plsc.bitcast(f, _u32)
  neg = b >= jnp.uint32(0x80000000)
  return jnp.where(neg, ~b, b | jnp.uint32(0x80000000))


def _popc(m):
  return jnp.sum(m.astype(_i32))


def _bit_of(kv, bitpos):
  return (jnp.right_shift(kv, bitpos.astype(_u32)) & jnp.uint32(1)) == jnp.uint32(1)


def _body(scores_hbm, bboxf, posef, kvisf, priorsf, stridef,
          obox, oscore, okpt, okscore,
          scores_v, cand0, cand1, selk0, seli0, selk1, seli1,
          score_s, ichunk, pidx, kidx, bidx, ridx, sidx,
          pose_t, kvis_t, bbox_t, prior_t, stride_t,
          obox_s, okpt_s, okscore_s, shared_idx, sem):
  c = lax.axis_index("c")
  s = lax.axis_index("s")
  row_local = s // 4
  b = c * 4 + row_local
  q = s % 4

  @pl.when(q == 0)
  def _phase_a():
    pltpu.sync_copy(scores_hbm.at[pl.ds(b * N, N)], scores_v)

    b31 = jnp.int32(31)

    def conv(v, cnt):
      f = scores_v[pl.ds(v * 16, 16)]
      u = _mono(f)
      cand0[pl.ds(v * 16, 16)] = u
      return cnt + _popc(_bit_of(u, b31))
    c_first = lax.fori_loop(0, NV, conv, jnp.int32(0))

    def select_step(src, dst, bitpos, state):
      remaining, prefix, ncand, c_cur = state
      go_right = c_cur >= remaining
      remaining = jnp.where(go_right, remaining, remaining - c_cur)
      prefix = jnp.where(
          go_right,
          prefix | lax.shift_left(jnp.uint32(1), bitpos.astype(_u32)),
          prefix)
      nbpos = jnp.maximum(bitpos - 1, 0)
      nv = (ncand + 15) // 16
      def cb(v, carry):
        wp, cnext = carry
        kv = src[pl.ds(v * 16, 16)]
        valid = (v * 16 + _IOTA()) < ncand
        m = valid & (_bit_of(kv, bitpos) == go_right)
        plsc.store_compressed(dst.at[pl.ds(wp, 16)], kv, mask=m)
        return wp + _popc(m), cnext + _popc(m & _bit_of(kv, nbpos))
      ncand, c_next = lax.fori_loop(0, nv, cb, (jnp.int32(0),) * 2)
      return remaining, prefix, ncand, c_next

    def radix_pair(i, state):
      state = select_step(cand0, cand1, 31 - 2 * i, state)
      state = select_step(cand1, cand0, 30 - 2 * i, state)
      return state

    r_final, thresh, _, _ = lax.fori_loop(
        0, 16, radix_pair,
        (jnp.int32(KEEP), jnp.uint32(0), jnp.int32(N), c_first))

    def sel(v, carry):
      wp, eq_before, ones0 = carry
      f = scores_v[pl.ds(v * 16, 16)]
      u = _mono(f)
      m_gt = u > thresh
      m_eq = u == thresh
      cs = plsc.cumsum(m_eq.astype(_i32))
      tot = _popc(m_eq)
      adj = jnp.where(jnp.max(cs) == tot, jnp.int32(1), jnp.int32(0))
      eq_rank = eq_before + cs - adj
      m = m_gt | (m_eq & (eq_rank < r_final))
      inv = ~u
      idxv = v * 16 + _IOTA()
      plsc.store_compressed(selk0.at[pl.ds(wp, 16)], inv, mask=m)
      plsc.store_compressed(seli0.at[pl.ds(wp, 16)], idxv, mask=m)
      ones0 = ones0 + _popc(m & ((inv & jnp.uint32(1)) == jnp.uint32(1)))
      return wp + _popc(m), eq_before + tot, ones0
    _, _, ones_b0 = lax.fori_loop(0, NV, sel, (jnp.int32(0),) * 3)

    def sortpass(sk, si, dk, di, bitpos, n_ones):
      wp1_init = jnp.int32(KEEP) - n_ones
      nbpos = jnp.minimum(bitpos + 1, 31)
      def sb(j, carry):
        wp0, wp1, cnext = carry
        kv = sk[pl.ds(j * 16, 16)]
        iv = si[pl.ds(j * 16, 16)]
        valid = (j * 16 + _IOTA()) < KEEP
        bit = _bit_of(kv, bitpos)
        m1 = valid & bit
        m0 = valid & ~bit
        plsc.store_compressed(dk.at[pl.ds(wp0, 16)], kv, mask=m0)
        plsc.store_compressed(di.at[pl.ds(wp0, 16)], iv, mask=m0)
        plsc.store_compressed(dk.at[pl.ds(wp1, 16)], kv, mask=m1)
        plsc.store_compressed(di.at[pl.ds(wp1, 16)], iv, mask=m1)
        nb = _bit_of(kv, nbpos)
        return wp0 + _popc(m0), wp1 + _popc(m1), cnext + _popc(valid & nb)
      _, _, cnext = lax.fori_loop(0, SELV, sb, (jnp.int32(0), wp1_init, jnp.int32(0)))
      return cnext

    def sort_pair(i, ones_in):
      ones_mid = sortpass(selk0, seli0, selk1, seli1, 2 * i, ones_in)
      return sortpass(selk1, seli1, selk0, seli0, 2 * i + 1, ones_mid)
    lax.fori_loop(0, 16, sort_pair, ones_b0)

    zeros16 = jnp.zeros((16,), _i32)
    seli0[pl.ds(KEEP, 16)] = zeros16
    seli0[pl.ds(1008, 16)] = zeros16
    pltpu.sync_copy(seli0, shared_idx.at[pl.ds(row_local * 1024, 1024)])

    def sc_out(j, carry):
      inv = selk0[pl.ds(j * 16, 16)]
      u = ~inv
      pos = u >= jnp.uint32(0x80000000)
      bits = jnp.where(pos, u ^ jnp.uint32(0x80000000), ~u)
      f = plsc.bitcast(bits, _f32)
      score_s[pl.ds(j * 16, 16)] = 1.0 / (1.0 + jnp.exp(-f))
      return carry
    lax.fori_loop(0, SELV, sc_out, 0)
    pltpu.sync_copy(score_s.at[pl.ds(0, KEEP)], oscore.at[pl.ds(b * KEEP, KEEP)])

  plsc.subcore_barrier()

  off = q * 256

  def half(h, hcarry):
    pltpu.sync_copy(shared_idx.at[pl.ds(row_local * 1024 + off + h * 128, 128)],
                    ichunk)

    def build(v, carry):
      sl = pl.ds(v * 16, 16)
      il = ichunk[sl]
      g = il + b * N
      p0 = g * (2 * K)
      for f in range(2 * K):
        pidx[f, sl] = p0 + f
      k0 = g * K
      for f in range(K):
        kidx[f, sl] = k0 + f
      b0 = g * 4
      for f in range(4):
        bidx[f, sl] = b0 + f
      r0 = il * 2
      for f in range(2):
        ridx[f, sl] = r0 + f
      sidx[0, sl] = il
      return carry
    lax.fori_loop(0, 8, build, 0)

    copies = []
    for f in range(2 * K):
      copies.append(pltpu.async_copy(
          posef.at[pidx.at[f]], pose_t.at[f], sem))
    for f in range(K):
      copies.append(pltpu.async_copy(
          kvisf.at[kidx.at[f]], kvis_t.at[f], sem))
    for f in range(4):
      copies.append(pltpu.async_copy(
          bboxf.at[bidx.at[f]], bbox_t.at[f], sem))
    for f in range(2):
      copies.append(pltpu.async_copy(
          priorsf.at[ridx.at[f]], prior_t.at[f], sem))
    copies.append(pltpu.async_copy(stridef.at[sidx.at[0]], stride_t.at[0], sem))
    for cp in copies:
      cp.wait()

    def decode(g16, carry):
      sl = pl.ds(g16 * 16, 16)
      e = g16 * 16 + _IOTA()
      e4 = e * 4
      ekp = e * (2 * K)
      eks = e * K
      bx = bbox_t[0, sl]
      by = bbox_t[1, sl]
      bw = bbox_t[2, sl]
      bh = bbox_t[3, sl]
      px = prior_t[0, sl]
      py = prior_t[1, sl]
      sv = stride_t[0, sl]
      cx = bx * sv + px
      cy = by * sv + py
      wx = jnp.exp(bw) * sv
      wy = jnp.exp(bh) * sv
      hx = wx * 0.5
      hy = wy * 0.5
      plsc.store_scatter(obox_s, [e4], cx - hx)
      plsc.store_scatter(obox_s, [e4 + 1], cy - hy)
      plsc.store_scatter(obox_s, [e4 + 2], cx + hx)
      plsc.store_scatter(obox_s, [e4 + 3], cy + hy)
      sx = wx * 0.625
      sy = wy * 0.625
      for k in range(K):
        ox = pose_t[2 * k, sl]
        oy = pose_t[2 * k + 1, sl]
        plsc.store_scatter(okpt_s, [ekp + 2 * k], cx + ox * sx)
        plsc.store_scatter(okpt_s, [ekp + 2 * k + 1], cy + oy * sy)
        vv = kvis_t[k, sl]
        plsc.store_scatter(okscore_s, [eks + k], 1.0 / (1.0 + jnp.exp(-vv)))
      return carry
    lax.fori_loop(0, 8, decode, 0)

    pos = b * KEEP + off + h * 128
    is_tail = (q == 3) & (h == 1)

    @pl.when(jnp.logical_not(is_tail))
    def _full():
      pltpu.sync_copy(obox_s, obox.at[pl.ds(pos * 4, 128 * 4)])
      pltpu.sync_copy(okpt_s, okpt.at[pl.ds(pos * 2 * K, 128 * 2 * K)])
      pltpu.sync_copy(okscore_s, okscore.at[pl.ds(pos * K, 128 * K)])

    @pl.when(is_tail)
    def _tail():
      pltpu.sync_copy(obox_s.at[pl.ds(0, 104 * 4)], obox.at[pl.ds(pos * 4, 104 * 4)])
      pltpu.sync_copy(okpt_s.at[pl.ds(0, 104 * 2 * K)],
                      okpt.at[pl.ds(pos * 2 * K, 104 * 2 * K)])
      pltpu.sync_copy(okscore_s.at[pl.ds(0, 104 * K)],
                      okscore.at[pl.ds(pos * K, 104 * K)])
    return hcarry

  lax.fori_loop(0, 2, half, 0)


@jax.jit
def kernel(cls_scores, bbox_preds, pose_vecs, kpt_vis, priors, stride):
  mesh = plsc.VectorSubcoreMesh(core_axis_name="c", subcore_axis_name="s",
                                num_cores=2, num_subcores=16)
  fn = pl.kernel(
      _body,
      out_type=[
          jax.ShapeDtypeStruct((B * KEEP * 4,), _f32),
          jax.ShapeDtypeStruct((B * KEEP,), _f32),
          jax.ShapeDtypeStruct((B * KEEP * 2 * K,), _f32),
          jax.ShapeDtypeStruct((B * KEEP * K,), _f32),
      ],
      mesh=mesh,
      compiler_params=pltpu.CompilerParams(needs_layout_passes=False,
                                           use_tc_tiling_on_sc=True),
      scratch_types=[
          pltpu.VMEM((N,), _f32),
          pltpu.VMEM((CAND,), _u32),
          pltpu.VMEM((CAND,), _u32),
          pltpu.VMEM((1024,), _u32),
          pltpu.VMEM((1024,), _i32),
          pltpu.VMEM((1024,), _u32),
          pltpu.VMEM((1024,), _i32),
          pltpu.VMEM((1008,), _f32),
          pltpu.VMEM((128,), _i32),
          pltpu.VMEM((2 * K, 128), _i32),
          pltpu.VMEM((K, 128), _i32),
          pltpu.VMEM((4, 128), _i32),
          pltpu.VMEM((2, 128), _i32),
          pltpu.VMEM((1, 128), _i32),
          pltpu.VMEM((2 * K, 128), _f32),
          pltpu.VMEM((K, 128), _f32),
          pltpu.VMEM((4, 128), _f32),
          pltpu.VMEM((2, 128), _f32),
          pltpu.VMEM((1, 128), _f32),
          pltpu.VMEM((128 * 4,), _f32),
          pltpu.VMEM((128 * 2 * K,), _f32),
          pltpu.VMEM((128 * K,), _f32),
          pltpu.VMEM_SHARED((4 * 1024,), _i32),
          pltpu.SemaphoreType.DMA,
      ],
  )
  obox, oscore, okpt, okscore = fn(
      cls_scores.reshape(B * N),
      bbox_preds.reshape(B * N * 4),
      pose_vecs.reshape(B * N * 2 * K),
      kpt_vis.reshape(B * N * K),
      priors.reshape(N * 2),
      stride,
  )
  return (obox.reshape(B, KEEP, 4),
          oscore.reshape(B, KEEP, 1),
          okpt.reshape(B, KEEP, K, 2),
          okscore.reshape(B, KEEP, K, 1))


# speedup vs baseline: 1.2229x; 1.2229x over previous
import functools

import jax
import jax.numpy as jnp
from jax import lax
from jax.experimental import pallas as pl
from jax.experimental.pallas import tpu as pltpu
from jax.experimental.pallas import tpu_sc as plsc

B = 8
N = 20000
K = 17
KEEP = 1000
NV = N // 16
SELV = 63
CAND = N + 32

_i32 = jnp.int32
_u32 = jnp.uint32
_f32 = jnp.float32

_IOTA = lambda: lax.iota(_i32, 16)


def _mono(f):
  b = plsc.bitcast(f, _u32)
  neg = b >= jnp.uint32(0x80000000)
  return jnp.where(neg, ~b, b | jnp.uint32(0x80000000))


def _popc(m):
  return plsc.all_reduce_population_count(m)[0]


def _bit_of(kv, bitpos):
  return (jnp.right_shift(kv, bitpos.astype(_u32)) & jnp.uint32(1)) == jnp.uint32(1)


def _body_a(scores_hbm, oidx, oscore,
            scores_v, cand0, cand1, selk0, seli0, selk1, seli1, score_s):
  c = lax.axis_index("c")
  s = lax.axis_index("s")
  b = c * 4 + s // 4
  q = s % 4

  @pl.when(q == 0)
  def _phase_a():
    pltpu.sync_copy(scores_hbm.at[pl.ds(b * N, N)], scores_v)

    b31 = jnp.int32(31)

    def conv(v, cnt):
      f = scores_v[pl.ds(v * 16, 16)]
      u = _mono(f)
      cand0[pl.ds(v * 16, 16)] = u
      return cnt + _popc(_bit_of(u, b31))
    c_first = lax.fori_loop(0, NV, conv, jnp.int32(0))

    def select_step(src, dst, bitpos, state):
      remaining, prefix, ncand, c_cur = state
      go_right = c_cur >= remaining
      remaining = jnp.where(go_right, remaining, remaining - c_cur)
      prefix = jnp.where(
          go_right,
          prefix | lax.shift_left(jnp.uint32(1), bitpos.astype(_u32)),
          prefix)
      nbpos = jnp.maximum(bitpos - 1, 0)
      nv = (ncand + 15) // 16
      def cb(v, carry):
        wp, cnext = carry
        kv = src[pl.ds(v * 16, 16)]
        valid = (v * 16 + _IOTA()) < ncand
        m = valid & (_bit_of(kv, bitpos) == go_right)
        plsc.store_compressed(dst.at[pl.ds(wp, 16)], kv, mask=m)
        return wp + _popc(m), cnext + _popc(m & _bit_of(kv, nbpos))
      ncand, c_next = lax.fori_loop(0, nv, cb, (jnp.int32(0),) * 2)
      return remaining, prefix, ncand, c_next

    def radix_pair(i, state):
      state = select_step(cand0, cand1, 31 - 2 * i, state)
      state = select_step(cand1, cand0, 30 - 2 * i, state)
      return state

    r_final, thresh, _, _ = lax.fori_loop(
        0, 16, radix_pair,
        (jnp.int32(KEEP), jnp.uint32(0), jnp.int32(N), c_first))

    def sel(v, carry):
      wp, we = carry
      f = scores_v[pl.ds(v * 16, 16)]
      u = _mono(f)
      m_gt = u > thresh
      m_eq = u == thresh
      idxv = v * 16 + _IOTA()
      plsc.store_compressed(selk0.at[pl.ds(wp, 16)], ~u, mask=m_gt)
      plsc.store_compressed(seli0.at[pl.ds(wp, 16)], idxv, mask=m_gt)
      eq_i = plsc.bitcast(idxv, _u32)
      plsc.store_compressed(cand1.at[pl.ds(we, 16)], eq_i, mask=m_eq)
      return wp + _popc(m_gt), we + _popc(m_eq)
    n_gt, _ = lax.fori_loop(0, NV, sel, (jnp.int32(0),) * 2)

    r_final = jnp.int32(KEEP) - n_gt
    inv_t = ~thresh
    def app(j, carry):
      valid = (j * 16 + _IOTA()) < r_final
      iv = plsc.bitcast(cand1[pl.ds(j * 16, 16)], _i32)
      plsc.store_compressed(selk0.at[pl.ds(n_gt + j * 16, 16)],
                            jnp.full((16,), inv_t, _u32), mask=valid)
      plsc.store_compressed(seli0.at[pl.ds(n_gt + j * 16, 16)], iv, mask=valid)
      return carry
    lax.fori_loop(0, (r_final + 15) // 16, app, 0)

    def cnt0(j, acc):
      kv = selk0[pl.ds(j * 16, 16)]
      valid = (j * 16 + _IOTA()) < KEEP
      return acc + _popc(valid & ((kv & jnp.uint32(1)) == jnp.uint32(1)))
    ones_b0 = lax.fori_loop(0, SELV, cnt0, jnp.int32(0))

    def sortpass(sk, si, dk, di, bitpos, n_ones):
      wp1_init = jnp.int32(KEEP) - n_ones
      nbpos = jnp.minimum(bitpos + 1, 31)
      def sb(j, carry):
        wp0, wp1, cnext = carry
        kv = sk[pl.ds(j * 16, 16)]
        iv = si[pl.ds(j * 16, 16)]
        valid = (j * 16 + _IOTA()) < KEEP
        vc = jnp.minimum(jnp.int32(16), jnp.int32(KEEP) - j * 16)
        bit = _bit_of(kv, bitpos)
        m1 = valid & bit
        m0 = valid & ~bit
        plsc.store_compressed(dk.at[pl.ds(wp0, 16)], kv, mask=m0)
        plsc.store_compressed(di.at[pl.ds(wp0, 16)], iv, mask=m0)
        plsc.store_compressed(dk.at[pl.ds(wp1, 16)], kv, mask=m1)
        plsc.store_compressed(di.at[pl.ds(wp1, 16)], iv, mask=m1)
        nb = _bit_of(kv, nbpos)
        c0 = _popc(m0)
        return wp0 + c0, wp1 + (vc - c0), cnext + _popc(valid & nb)
      _, _, cnext = lax.fori_loop(0, SELV, sb, (jnp.int32(0), wp1_init, jnp.int32(0)))
      return cnext

    def sort_pair(i, ones_in):
      ones_mid = sortpass(selk0, seli0, selk1, seli1, 2 * i, ones_in)
      return sortpass(selk1, seli1, selk0, seli0, 2 * i + 1, ones_mid)
    lax.fori_loop(0, 16, sort_pair, ones_b0)

    zeros16 = jnp.zeros((16,), _i32)
    seli0[pl.ds(KEEP, 16)] = zeros16
    seli0[pl.ds(1008, 16)] = zeros16
    pltpu.sync_copy(seli0, oidx.at[pl.ds(b * 1024, 1024)])

    def sc_out(j, carry):
      inv = selk0[pl.ds(j * 16, 16)]
      u = ~inv
      pos = u >= jnp.uint32(0x80000000)
      bits = jnp.where(pos, u ^ jnp.uint32(0x80000000), ~u)
      f = plsc.bitcast(bits, _f32)
      score_s[pl.ds(j * 16, 16)] = 1.0 / (1.0 + jnp.exp(-f))
      return carry
    lax.fori_loop(0, SELV, sc_out, 0)
    pltpu.sync_copy(score_s.at[pl.ds(0, KEEP)], oscore.at[pl.ds(b * KEEP, KEEP)])


def _body_b(idx_hbm, bboxf, posef, kvisf, priorsf, stridef,
            obox, okpt, okscore,
            ichunk, pidx, kidx, bidx, ridx, sidx,
            pose_t, kvis_t, bbox_t, prior_t, stride_t,
            obox_s, okpt_s, okscore_s, sem):
  c = lax.axis_index("c")
  s = lax.axis_index("s")
  b = c * 4 + s // 4
  q = s % 4
  off = q * 256

  def half(h, hcarry):
    pltpu.sync_copy(idx_hbm.at[pl.ds(b * 1024 + off + h * 128, 128)], ichunk)

    def build(v, carry):
      sl = pl.ds(v * 16, 16)
      il = ichunk[sl]
      g = il + b * N
      p0 = g * (2 * K)
      for f in range(2 * K):
        pidx[f, sl] = p0 + f
      k0 = g * K
      for f in range(K):
        kidx[f, sl] = k0 + f
      b0 = g * 4
      for f in range(4):
        bidx[f, sl] = b0 + f
      r0 = il * 2
      for f in range(2):
        ridx[f, sl] = r0 + f
      sidx[0, sl] = il
      return carry
    lax.fori_loop(0, 8, build, 0)

    copies = []
    for f in range(2 * K):
      copies.append(pltpu.async_copy(posef.at[pidx.at[f]], pose_t.at[f], sem))
    for f in range(K):
      copies.append(pltpu.async_copy(kvisf.at[kidx.at[f]], kvis_t.at[f], sem))
    for f in range(4):
      copies.append(pltpu.async_copy(bboxf.at[bidx.at[f]], bbox_t.at[f], sem))
    for f in range(2):
      copies.append(pltpu.async_copy(priorsf.at[ridx.at[f]], prior_t.at[f], sem))
    copies.append(pltpu.async_copy(stridef.at[sidx.at[0]], stride_t.at[0], sem))
    for cp in copies:
      cp.wait()

    def decode(g16, carry):
      sl = pl.ds(g16 * 16, 16)
      e = g16 * 16 + _IOTA()
      e4 = e * 4
      ekp = e * (2 * K)
      eks = e * K
      bx = bbox_t[0, sl]
      by = bbox_t[1, sl]
      bw = bbox_t[2, sl]
      bh = bbox_t[3, sl]
      px = prior_t[0, sl]
      py = prior_t[1, sl]
      sv = stride_t[0, sl]
      cx = bx * sv + px
      cy = by * sv + py
      wx = jnp.exp(bw) * sv
      wy = jnp.exp(bh) * sv
      hx = wx * 0.5
      hy = wy * 0.5
      plsc.store_scatter(obox_s, [e4], cx - hx)
      plsc.store_scatter(obox_s, [e4 + 1], cy - hy)
      plsc.store_scatter(obox_s, [e4 + 2], cx + hx)
      plsc.store_scatter(obox_s, [e4 + 3], cy + hy)
      sx = wx * 0.625
      sy = wy * 0.625
      for k in range(K):
        ox = pose_t[2 * k, sl]
        oy = pose_t[2 * k + 1, sl]
        plsc.store_scatter(okpt_s, [ekp + 2 * k], cx + ox * sx)
        plsc.store_scatter(okpt_s, [ekp + 2 * k + 1], cy + oy * sy)
        vv = kvis_t[k, sl]
        plsc.store_scatter(okscore_s, [eks + k], 1.0 / (1.0 + jnp.exp(-vv)))
      return carry
    lax.fori_loop(0, 8, decode, 0)

    pos = b * KEEP + off + h * 128
    is_tail = (q == 3) & (h == 1)

    @pl.when(jnp.logical_not(is_tail))
    def _full():
      pltpu.sync_copy(obox_s, obox.at[pl.ds(pos * 4, 128 * 4)])
      pltpu.sync_copy(okpt_s, okpt.at[pl.ds(pos * 2 * K, 128 * 2 * K)])
      pltpu.sync_copy(okscore_s, okscore.at[pl.ds(pos * K, 128 * K)])

    @pl.when(is_tail)
    def _tail():
      pltpu.sync_copy(obox_s.at[pl.ds(0, 104 * 4)], obox.at[pl.ds(pos * 4, 104 * 4)])
      pltpu.sync_copy(okpt_s.at[pl.ds(0, 104 * 2 * K)],
                      okpt.at[pl.ds(pos * 2 * K, 104 * 2 * K)])
      pltpu.sync_copy(okscore_s.at[pl.ds(0, 104 * K)],
                      okscore.at[pl.ds(pos * K, 104 * K)])
    return hcarry

  lax.fori_loop(0, 2, half, 0)


@jax.jit
def kernel(cls_scores, bbox_preds, pose_vecs, kpt_vis, priors, stride):
  mesh = plsc.VectorSubcoreMesh(core_axis_name="c", subcore_axis_name="s",
                                num_cores=2, num_subcores=16)
  cp = pltpu.CompilerParams(needs_layout_passes=False,
                            use_tc_tiling_on_sc=False)
  fn_a = pl.kernel(
      _body_a,
      out_type=[
          jax.ShapeDtypeStruct((B * 1024,), _i32),
          jax.ShapeDtypeStruct((B * KEEP,), _f32),
      ],
      mesh=mesh,
      compiler_params=cp,
      scratch_types=[
          pltpu.VMEM((N,), _f32),
          pltpu.VMEM((CAND,), _u32),
          pltpu.VMEM((CAND,), _u32),
          pltpu.VMEM((1024,), _u32),
          pltpu.VMEM((1024,), _i32),
          pltpu.VMEM((1024,), _u32),
          pltpu.VMEM((1024,), _i32),
          pltpu.VMEM((1008,), _f32),
      ],
  )
  fn_b = pl.kernel(
      _body_b,
      out_type=[
          jax.ShapeDtypeStruct((B * KEEP * 4,), _f32),
          jax.ShapeDtypeStruct((B * KEEP * 2 * K,), _f32),
          jax.ShapeDtypeStruct((B * KEEP * K,), _f32),
      ],
      mesh=mesh,
      compiler_params=cp,
      scratch_types=[
          pltpu.VMEM((128,), _i32),
          pltpu.VMEM((2 * K, 128), _i32),
          pltpu.VMEM((K, 128), _i32),
          pltpu.VMEM((4, 128), _i32),
          pltpu.VMEM((2, 128), _i32),
          pltpu.VMEM((1, 128), _i32),
          pltpu.VMEM((2 * K, 128), _f32),
          pltpu.VMEM((K, 128), _f32),
          pltpu.VMEM((4, 128), _f32),
          pltpu.VMEM((2, 128), _f32),
          pltpu.VMEM((1, 128), _f32),
          pltpu.VMEM((128 * 4,), _f32),
          pltpu.VMEM((128 * 2 * K,), _f32),
          pltpu.VMEM((128 * K,), _f32),
          pltpu.SemaphoreType.DMA,
      ],
  )
  oidx, oscore = fn_a(cls_scores.reshape(B * N))
  obox, okpt, okscore = fn_b(
      oidx,
      bbox_preds.reshape(B * N * 4),
      pose_vecs.reshape(B * N * 2 * K),
      kpt_vis.reshape(B * N * K),
      priors.reshape(N * 2),
      stride,
  )
  return (obox.reshape(B, KEEP, 4),
          oscore.reshape(B, KEEP, 1),
          okpt.reshape(B, KEEP, K, 2),
          okscore.reshape(B, KEEP, K, 1))


# speedup vs baseline: 1.6457x; 1.3457x over previous
import functools

import jax
import jax.numpy as jnp
from jax import lax
from jax.experimental import pallas as pl
from jax.experimental.pallas import tpu as pltpu
from jax.experimental.pallas import tpu_sc as plsc

B = 8
N = 20000
K = 17
KEEP = 1000
NV = N // 16
SELV = 63
CAND = N + 32

_i32 = jnp.int32
_u32 = jnp.uint32
_f32 = jnp.float32

_IOTA = lambda: lax.iota(_i32, 16)


def _mono(f):
  b = plsc.bitcast(f, _u32)
  neg = b >= jnp.uint32(0x80000000)
  return jnp.where(neg, ~b, b | jnp.uint32(0x80000000))


def _popc(m):
  return plsc.all_reduce_population_count(m)[0]


def _bit_of(kv, bitpos):
  return (jnp.right_shift(kv, bitpos.astype(_u32)) & jnp.uint32(1)) == jnp.uint32(1)


def _body_a(scores_hbm, oidx, oscore,
            scores_v, cand0, cand1, selk0, seli0, selk1, seli1, score_s):
  c = lax.axis_index("c")
  s = lax.axis_index("s")
  b = c * 4 + s // 4
  q = s % 4

  @pl.when(q == 0)
  def _phase_a():
    pltpu.sync_copy(scores_hbm.at[pl.ds(b * N, N)], scores_v)

    b31 = jnp.int32(31)

    def conv(v, cnt):
      f = scores_v[pl.ds(v * 16, 16)]
      u = _mono(f)
      cand0[pl.ds(v * 16, 16)] = u
      return cnt + _popc(_bit_of(u, b31))
    c_first = lax.fori_loop(0, NV, conv, jnp.int32(0))

    def select_step(src, dst, bitpos, state):
      remaining, prefix, ncand, c_cur = state
      go_right = c_cur >= remaining
      remaining = jnp.where(go_right, remaining, remaining - c_cur)
      prefix = jnp.where(
          go_right,
          prefix | lax.shift_left(jnp.uint32(1), bitpos.astype(_u32)),
          prefix)
      nbpos = jnp.maximum(bitpos - 1, 0)
      nv = (ncand + 15) // 16
      def cb(v, carry):
        wp, cnext = carry
        kv = src[pl.ds(v * 16, 16)]
        valid = (v * 16 + _IOTA()) < ncand
        m = valid & (_bit_of(kv, bitpos) == go_right)
        plsc.store_compressed(dst.at[pl.ds(wp, 16)], kv, mask=m)
        return wp + _popc(m), cnext + _popc(m & _bit_of(kv, nbpos))
      ncand, c_next = lax.fori_loop(0, nv, cb, (jnp.int32(0),) * 2)
      return remaining, prefix, ncand, c_next

    def radix_pair(i, state):
      state = select_step(cand0, cand1, 31 - 2 * i, state)
      state = select_step(cand1, cand0, 30 - 2 * i, state)
      return state

    r_final, thresh, _, _ = lax.fori_loop(
        0, 16, radix_pair,
        (jnp.int32(KEEP), jnp.uint32(0), jnp.int32(N), c_first))

    def sel(v, carry):
      wp, we = carry
      f = scores_v[pl.ds(v * 16, 16)]
      u = _mono(f)
      m_gt = u > thresh
      m_eq = u == thresh
      idxv = v * 16 + _IOTA()
      plsc.store_compressed(selk0.at[pl.ds(wp, 16)], ~u, mask=m_gt)
      plsc.store_compressed(seli0.at[pl.ds(wp, 16)], idxv, mask=m_gt)
      eq_i = plsc.bitcast(idxv, _u32)
      plsc.store_compressed(cand1.at[pl.ds(we, 16)], eq_i, mask=m_eq)
      return wp + _popc(m_gt), we + _popc(m_eq)
    n_gt, _ = lax.fori_loop(0, NV, sel, (jnp.int32(0),) * 2)

    r_final = jnp.int32(KEEP) - n_gt
    inv_t = ~thresh
    def app(j, carry):
      valid = (j * 16 + _IOTA()) < r_final
      iv = plsc.bitcast(cand1[pl.ds(j * 16, 16)], _i32)
      plsc.store_compressed(selk0.at[pl.ds(n_gt + j * 16, 16)],
                            jnp.full((16,), inv_t, _u32), mask=valid)
      plsc.store_compressed(seli0.at[pl.ds(n_gt + j * 16, 16)], iv, mask=valid)
      return carry
    lax.fori_loop(0, (r_final + 15) // 16, app, 0)

    def cnt0(j, acc):
      kv = selk0[pl.ds(j * 16, 16)]
      valid = (j * 16 + _IOTA()) < KEEP
      return acc + _popc(valid & ((kv & jnp.uint32(1)) == jnp.uint32(1)))
    ones_b0 = lax.fori_loop(0, SELV, cnt0, jnp.int32(0))

    def sortpass(sk, si, dk, di, bitpos, n_ones):
      wp1_init = jnp.int32(KEEP) - n_ones
      nbpos = jnp.minimum(bitpos + 1, 31)
      def sb(j, carry):
        wp0, wp1, cnext = carry
        kv = sk[pl.ds(j * 16, 16)]
        iv = si[pl.ds(j * 16, 16)]
        valid = (j * 16 + _IOTA()) < KEEP
        vc = jnp.minimum(jnp.int32(16), jnp.int32(KEEP) - j * 16)
        bit = _bit_of(kv, bitpos)
        m1 = valid & bit
        m0 = valid & ~bit
        plsc.store_compressed(dk.at[pl.ds(wp0, 16)], kv, mask=m0)
        plsc.store_compressed(di.at[pl.ds(wp0, 16)], iv, mask=m0)
        plsc.store_compressed(dk.at[pl.ds(wp1, 16)], kv, mask=m1)
        plsc.store_compressed(di.at[pl.ds(wp1, 16)], iv, mask=m1)
        nb = _bit_of(kv, nbpos)
        c0 = _popc(m0)
        return wp0 + c0, wp1 + (vc - c0), cnext + _popc(valid & nb)
      _, _, cnext = lax.fori_loop(0, SELV, sb, (jnp.int32(0), wp1_init, jnp.int32(0)))
      return cnext

    def sort_pair(i, ones_in):
      ones_mid = sortpass(selk0, seli0, selk1, seli1, 2 * i, ones_in)
      return sortpass(selk1, seli1, selk0, seli0, 2 * i + 1, ones_mid)
    lax.fori_loop(0, 16, sort_pair, ones_b0)

    zeros16 = jnp.zeros((16,), _i32)
    seli0[pl.ds(KEEP, 16)] = zeros16
    seli0[pl.ds(1008, 16)] = zeros16
    pltpu.sync_copy(seli0, oidx.at[pl.ds(b * 1024, 1024)])

    def sc_out(j, carry):
      inv = selk0[pl.ds(j * 16, 16)]
      u = ~inv
      pos = u >= jnp.uint32(0x80000000)
      bits = jnp.where(pos, u ^ jnp.uint32(0x80000000), ~u)
      f = plsc.bitcast(bits, _f32)
      score_s[pl.ds(j * 16, 16)] = 1.0 / (1.0 + jnp.exp(-f))
      return carry
    lax.fori_loop(0, SELV, sc_out, 0)
    pltpu.sync_copy(score_s.at[pl.ds(0, KEEP)], oscore.at[pl.ds(b * KEEP, KEEP)])


def _body_b(idx_hbm, bboxf, posef, kvisf, priorsf, stridef,
            obox, okpt, okscore,
            ichunk, pidx, kidx, bidx, ridx, sidx,
            pose_t, kvis_t, bbox_t, prior_t, stride_t,
            obox_s, okpt_s, okscore_s, sem):
  c = lax.axis_index("c")
  s = lax.axis_index("s")
  b = c * 4 + s // 4
  q = s % 4
  off = q * 256

  def half(h, hcarry):
    pltpu.sync_copy(idx_hbm.at[pl.ds(b * 1024 + off + h * 128, 128)], ichunk)

    def build(v, carry):
      sl = pl.ds(v * 16, 16)
      il = ichunk[sl]
      g = il + b * N
      p0 = g * (2 * K)
      for f in range(2 * K):
        pidx[f, sl] = p0 + f
      k0 = g * K
      for f in range(K):
        kidx[f, sl] = k0 + f
      b0 = g * 4
      for f in range(4):
        bidx[f, sl] = b0 + f
      r0 = il * 2
      for f in range(2):
        ridx[f, sl] = r0 + f
      sidx[0, sl] = il
      return carry
    lax.fori_loop(0, 8, build, 0)

    copies = []
    for f in range(2 * K):
      copies.append(pltpu.async_copy(posef.at[pidx.at[f]], pose_t.at[f], sem))
    for f in range(K):
      copies.append(pltpu.async_copy(kvisf.at[kidx.at[f]], kvis_t.at[f], sem))
    for f in range(4):
      copies.append(pltpu.async_copy(bboxf.at[bidx.at[f]], bbox_t.at[f], sem))
    for f in range(2):
      copies.append(pltpu.async_copy(priorsf.at[ridx.at[f]], prior_t.at[f], sem))
    copies.append(pltpu.async_copy(stridef.at[sidx.at[0]], stride_t.at[0], sem))
    for cp in copies:
      cp.wait()

    def decode(g16, carry):
      sl = pl.ds(g16 * 16, 16)
      e = g16 * 16 + _IOTA()
      col = lambda cc: jnp.full((16,), cc, _i32)
      bx = bbox_t[0, sl]
      by = bbox_t[1, sl]
      bw = bbox_t[2, sl]
      bh = bbox_t[3, sl]
      px = prior_t[0, sl]
      py = prior_t[1, sl]
      sv = stride_t[0, sl]
      cx = bx * sv + px
      cy = by * sv + py
      wx = jnp.exp(bw) * sv
      wy = jnp.exp(bh) * sv
      hx = wx * 0.5
      hy = wy * 0.5
      plsc.store_scatter(obox_s, [e, col(0)], cx - hx)
      plsc.store_scatter(obox_s, [e, col(1)], cy - hy)
      plsc.store_scatter(obox_s, [e, col(2)], cx + hx)
      plsc.store_scatter(obox_s, [e, col(3)], cy + hy)
      sx = wx * 0.625
      sy = wy * 0.625
      for k in range(K):
        ox = pose_t[2 * k, sl]
        oy = pose_t[2 * k + 1, sl]
        plsc.store_scatter(okpt_s, [e, col(2 * k)], cx + ox * sx)
        plsc.store_scatter(okpt_s, [e, col(2 * k + 1)], cy + oy * sy)
        vv = kvis_t[k, sl]
        plsc.store_scatter(okscore_s, [e, col(k)], 1.0 / (1.0 + jnp.exp(-vv)))
      return carry
    lax.fori_loop(0, 8, decode, 0)

    pos = b * KEEP + off + h * 128
    is_tail = (q == 3) & (h == 1)

    @pl.when(jnp.logical_not(is_tail))
    def _full():
      pltpu.sync_copy(obox_s, obox.at[pl.ds(pos, 128)])
      pltpu.sync_copy(okpt_s, okpt.at[pl.ds(pos, 128)])
      pltpu.sync_copy(okscore_s, okscore.at[pl.ds(pos, 128)])

    @pl.when(is_tail)
    def _tail():
      pltpu.sync_copy(obox_s.at[pl.ds(0, 104)], obox.at[pl.ds(pos, 104)])
      pltpu.sync_copy(okpt_s.at[pl.ds(0, 104)], okpt.at[pl.ds(pos, 104)])
      pltpu.sync_copy(okscore_s.at[pl.ds(0, 104)], okscore.at[pl.ds(pos, 104)])
    return hcarry

  lax.fori_loop(0, 2, half, 0)


@jax.jit
def kernel(cls_scores, bbox_preds, pose_vecs, kpt_vis, priors, stride):
  mesh = plsc.VectorSubcoreMesh(core_axis_name="c", subcore_axis_name="s",
                                num_cores=2, num_subcores=16)
  cp = pltpu.CompilerParams(needs_layout_passes=False,
                            use_tc_tiling_on_sc=False)
  fn_a = pl.kernel(
      _body_a,
      out_type=[
          jax.ShapeDtypeStruct((B * 1024,), _i32),
          jax.ShapeDtypeStruct((B * KEEP,), _f32),
      ],
      mesh=mesh,
      compiler_params=cp,
      scratch_types=[
          pltpu.VMEM((N,), _f32),
          pltpu.VMEM((CAND,), _u32),
          pltpu.VMEM((CAND,), _u32),
          pltpu.VMEM((1024,), _u32),
          pltpu.VMEM((1024,), _i32),
          pltpu.VMEM((1024,), _u32),
          pltpu.VMEM((1024,), _i32),
          pltpu.VMEM((1008,), _f32),
      ],
  )
  fn_b = pl.kernel(
      _body_b,
      out_type=[
          jax.ShapeDtypeStruct((B * KEEP, 4), _f32),
          jax.ShapeDtypeStruct((B * KEEP, 2 * K), _f32),
          jax.ShapeDtypeStruct((B * KEEP, K), _f32),
      ],
      mesh=mesh,
      compiler_params=cp,
      scratch_types=[
          pltpu.VMEM((128,), _i32),
          pltpu.VMEM((2 * K, 128), _i32),
          pltpu.VMEM((K, 128), _i32),
          pltpu.VMEM((4, 128), _i32),
          pltpu.VMEM((2, 128), _i32),
          pltpu.VMEM((1, 128), _i32),
          pltpu.VMEM((2 * K, 128), _f32),
          pltpu.VMEM((K, 128), _f32),
          pltpu.VMEM((4, 128), _f32),
          pltpu.VMEM((2, 128), _f32),
          pltpu.VMEM((1, 128), _f32),
          pltpu.VMEM((128, 4), _f32),
          pltpu.VMEM((128, 2 * K), _f32),
          pltpu.VMEM((128, K), _f32),
          pltpu.SemaphoreType.DMA,
      ],
  )
  oidx, oscore = fn_a(cls_scores.reshape(B * N))
  obox, okpt, okscore = fn_b(
      oidx,
      bbox_preds.reshape(B * N * 4),
      pose_vecs.reshape(B * N * 2 * K),
      kpt_vis.reshape(B * N * K),
      priors.reshape(N * 2),
      stride,
  )
  return (obox.reshape(B, KEEP, 4),
          oscore.reshape(B, KEEP, 1),
          okpt.reshape(B, KEEP, K, 2),
          okscore.reshape(B, KEEP, K, 1))
